# baseline (speedup 1.0000x reference)
import functools

import jax
import jax.numpy as jnp
from jax import lax
from jax.experimental import pallas as pl
from jax.experimental.pallas import tpu as pltpu
from jax.experimental.pallas import tpu_sc as plsc

DIM = 32
NN = 16
NREL = 64
NC = 2
NS = 16
NW = NC * NS
CH = 4
L = 16


def _bc(x, dtype=jnp.float32):
    return lax.broadcast(x, (L,))


QN = CH * NN
NB = CH * NN * NN // 128


def _sc_body(u_flat, adj_ent, adj_rel, ent_table, relT, pre0, pre1, r0o,
             uf, e1_all, r0_all, v0_all, relT_s, e1f_all,
             e2c2, r1c2, v1c2, e2fs2, v2c2, wbuf2, pre0c2, pre1c2,
             g1sem, g2sems, g3sems):
    wid = lax.axis_index("s") * NC + lax.axis_index("c")
    qw = u_flat.shape[0] // NW
    nch = qw // CH
    w0 = wid * qw

    pltpu.sync_copy(relT, relT_s)
    pltpu.sync_copy(u_flat.at[pl.ds(w0, qw)], uf)
    a1 = pltpu.async_copy(adj_ent.at[uf], e1_all, g1sem)
    a2 = pltpu.async_copy(adj_rel.at[uf], r0_all, g1sem)
    a3 = pltpu.async_copy(ent_table.at[uf], v0_all, g1sem)
    a1.wait(); a2.wait(); a3.wait()
    pltpu.sync_copy(r0_all, r0o.at[pl.ds(w0, qw)])
    for k in range(qw):
        e1f_all[pl.ds(k * NN, NN)] = e1_all[k]

    def g2_issue(ci, p):
        idx = e1f_all.at[pl.ds(ci * QN, QN)]
        pltpu.async_copy(adj_ent.at[idx], e2c2[p], g2sems[p])
        pltpu.async_copy(adj_rel.at[idx], r1c2[p], g2sems[p])
        pltpu.async_copy(ent_table.at[idx], v1c2[p], g2sems[p])

    def g2_wait(p):
        idx = e1f_all.at[pl.ds(0, QN)]
        pltpu.make_async_copy(adj_ent.at[idx], e2c2[p], g2sems[p]).wait()
        pltpu.make_async_copy(adj_rel.at[idx], r1c2[p], g2sems[p]).wait()
        pltpu.make_async_copy(ent_table.at[idx], v1c2[p], g2sems[p]).wait()

    def g3_issue(p):
        for k in range(NB):
            pltpu.async_copy(ent_table.at[e2fs2[p][k]],
                             v2c2[p].at[pl.ds(k * 128, 128)], g3sems[p])

    def g3_wait(p):
        for k in range(NB):
            pltpu.make_async_copy(ent_table.at[e2fs2[p][k]],
                                  v2c2[p].at[pl.ds(k * 128, 128)],
                                  g3sems[p]).wait()

    def score(r_row, va, vb):
        s = [jnp.zeros((L,), jnp.float32) for _ in range(4)]
        for d in range(L):
            k = (d & 1) * 2
            s[k] = s[k] + plsc.load_gather(relT_s,
                                           [r_row + d * NREL]) * _bc(va[d])
            s[k + 1] = s[k + 1] + plsc.load_gather(
                relT_s, [r_row + (L + d) * NREL]) * _bc(vb[d])
        return (s[0] + s[1]) + (s[2] + s[3])

    def xstage(ci, p):
        r1c, v1c, wbuf, pre1c, pre0c = (
            r1c2[p], v1c2[p], wbuf2[p], pre1c2[p], pre0c2[p])

        def jbody(lr, carry2):
            r1row = r1c[lr]
            v1a = v1c[lr, 0:L]
            v1b = v1c[lr, L:DIM]
            s = score(r1row, v1a, v1b)
            m = jnp.max(s)
            e = jnp.exp(s - _bc(m))
            inv = jnp.ones((L,), jnp.float32) / _bc(jnp.sum(e))
            wbuf[lr] = e * inv
            pre1c[lr, 0:L] = v1a
            pre1c[lr, L:DIM] = v1b
            return carry2

        lax.fori_loop(0, QN, jbody, 0)

        for q in range(CH):
            qg = ci * CH + q
            r0row = r0_all[qg]
            v0a = v0_all[qg, 0:L]
            v0b = v0_all[qg, L:DIM]
            s = score(r0row, v0a, v0b)
            m = jnp.max(s)
            e = jnp.exp(s - _bc(m))
            ssum = jnp.sum(e)
            acc = [jnp.zeros((L,), jnp.float32) for _ in range(4)]
            for n in range(NN):
                wn = _bc(e[n])
                k = (n & 1) * 2
                acc[k] = acc[k] + wn * v1c[q * NN + n, 0:L]
                acc[k + 1] = acc[k + 1] + wn * v1c[q * NN + n, L:DIM]
            inv = jnp.ones((L,), jnp.float32) / _bc(ssum)
            pre0c[q, 0:L] = v0_all[qg, 0:L] + (acc[0] + acc[2]) * inv
            pre0c[q, L:DIM] = v0_all[qg, L:DIM] + (acc[1] + acc[3]) * inv

    def flatten_e2(p):
        e2c = e2c2[p]
        for k in range(QN):
            e2fs2[p][k // 8][pl.ds((k % 8) * NN, NN)] = e2c[k]

    def ystage(ci, p):
        g3_wait(p)
        v2c, wbuf, pre1c, pre0c = v2c2[p], wbuf2[p], pre1c2[p], pre0c2[p]

        def jbody(lr, carry2):
            w = wbuf[lr]
            base = lr * NN
            acc = [jnp.zeros((L,), jnp.float32) for _ in range(4)]
            for n in range(NN):
                wn = _bc(w[n])
                k = (n & 1) * 2
                acc[k] = acc[k] + wn * v2c[base + n, 0:L]
                acc[k + 1] = acc[k + 1] + wn * v2c[base + n, L:DIM]
            pre1c[lr, 0:L] = pre1c[lr, 0:L] + (acc[0] + acc[2])
            pre1c[lr, L:DIM] = pre1c[lr, L:DIM] + (acc[1] + acc[3])
            return carry2

        lax.fori_loop(0, QN, jbody, 0)
        q0 = w0 + ci * CH
        pltpu.sync_copy(pre0c, pre0.at[pl.ds(q0, CH)])
        pltpu.sync_copy(pre1c, pre1.at[pl.ds(q0 * NN, QN)])

    g2_issue(0, 0)

    def outer(io, carry):
        for p in (0, 1):
            ci = 2 * io + p
            g2_wait(p)
            xstage(ci, p)
            flatten_e2(p)
            g3_issue(p)

            @pl.when(ci < nch - 1)
            def _():
                g2_issue(ci + 1, 1 - p)

            @pl.when(ci > 0)
            def _():
                ystage(ci - 1, 1 - p)
        return carry

    lax.fori_loop(0, nch // 2, outer, 0)
    ystage(nch - 1, 1)


def _sc_call(u, adj_ent, adj_rel, ent_table, relT):
    B = u.shape[0]
    mesh = plsc.VectorSubcoreMesh(core_axis_name="c", subcore_axis_name="s",
                                  num_cores=NC, num_subcores=NS)
    qw = B // NW
    body = functools.partial(
        pl.kernel,
        out_type=(
            jax.ShapeDtypeStruct((B, DIM), jnp.float32),
            jax.ShapeDtypeStruct((B * NN, DIM), jnp.float32),
            jax.ShapeDtypeStruct((B, NN), jnp.int32),
        ),
        mesh=mesh,
        compiler_params=pltpu.CompilerParams(needs_layout_passes=False,
                                             use_tc_tiling_on_sc=False),
        scratch_types=[
            pltpu.VMEM((qw,), jnp.int32),
            pltpu.VMEM((qw, NN), jnp.int32),
            pltpu.VMEM((qw, NN), jnp.int32),
            pltpu.VMEM((qw, DIM), jnp.float32),
            pltpu.VMEM((DIM * NREL,), jnp.float32),
            pltpu.VMEM((qw * NN,), jnp.int32),
            [pltpu.VMEM((QN, NN), jnp.int32)] * 2,
            [pltpu.VMEM((QN, NN), jnp.int32)] * 2,
            [pltpu.VMEM((QN, DIM), jnp.float32)] * 2,
            [[pltpu.VMEM((128,), jnp.int32)
              for _ in range(NB)] for _ in range(2)],
            [pltpu.VMEM((CH * NN * NN, DIM), jnp.float32)] * 2,
            [pltpu.VMEM((QN, L), jnp.float32)] * 2,
            [pltpu.VMEM((CH, DIM), jnp.float32)] * 2,
            [pltpu.VMEM((QN, DIM), jnp.float32)] * 2,
            pltpu.SemaphoreType.DMA,
            [pltpu.SemaphoreType.DMA] * 2,
            [pltpu.SemaphoreType.DMA] * 2,
        ],
    )(_sc_body)
    return body(u, adj_ent, adj_rel, ent_table, relT)


BK2 = 256


def _tail_kernel(pre0_ref, pre1_ref, r0_ref, relT_ref, w_ref, b_ref, out_ref):
    bk = pre0_ref.shape[0]
    W = w_ref[...]
    b = b_ref[...]
    h0 = jax.nn.sigmoid(pre0_ref[...] @ W + b)
    h1 = jax.nn.sigmoid(pre1_ref[...] @ W + b)
    sp = h0 @ relT_ref[...]
    r0 = r0_ref[...]
    rid = lax.broadcasted_iota(jnp.int32, (bk, NN, NREL), 2)
    onehot = (r0[:, :, None] == rid).astype(jnp.float32)
    sh = jnp.sum(onehot * sp[:, None, :], axis=-1)
    sh = sh - jnp.max(sh, axis=-1, keepdims=True)
    eh = jnp.exp(sh)
    wh = eh / jnp.sum(eh, axis=-1, keepdims=True)
    h1g = h1.reshape(bk, NN, DIM)
    aggh = jnp.sum(wh[:, :, None] * h1g, axis=1)
    out_ref[...] = jnp.tanh((h0 + aggh) @ W + b)


def kernel(drug_entity_list, adj_ent, adj_rel, drug_table, ent_table, rel_table, W, b):
    B = drug_entity_list.shape[0]
    relT = rel_table.T.reshape(-1)
    pre0, pre1, r0 = _sc_call(drug_entity_list, adj_ent, adj_rel, ent_table,
                              relT)
    grid = B // BK2
    out = pl.pallas_call(
        _tail_kernel,
        grid=(grid,),
        in_specs=[
            pl.BlockSpec((BK2, DIM), lambda i: (i, 0)),
            pl.BlockSpec((BK2 * NN, DIM), lambda i: (i, 0)),
            pl.BlockSpec((BK2, NN), lambda i: (i, 0)),
            pl.BlockSpec((DIM, NREL), lambda i: (0, 0)),
            pl.BlockSpec((DIM, DIM), lambda i: (0, 0)),
            pl.BlockSpec((1, DIM), lambda i: (0, 0)),
        ],
        out_specs=pl.BlockSpec((BK2, DIM), lambda i: (i, 0)),
        out_shape=jax.ShapeDtypeStruct((B, DIM), jnp.float32),
    )(pre0, pre1, r0, rel_table.T, W, b.reshape(1, DIM))
    return out


# speedup vs baseline: 1.0083x; 1.0083x over previous
import functools

import jax
import jax.numpy as jnp
from jax import lax
from jax.experimental import pallas as pl
from jax.experimental.pallas import tpu as pltpu
from jax.experimental.pallas import tpu_sc as plsc

DIM = 32
NN = 16
NREL = 64
NC = 2
NS = 16
NW = NC * NS
CH = 4
L = 16


def _bc(x, dtype=jnp.float32):
    return lax.broadcast(x, (L,))


QN = CH * NN
NB = CH * NN * NN // 128


def _sc_body(u_flat, adj_ent, adj_rel, ent_table, relT, pre0, pre1, r0o,
             uf, e1_all, r0_all, v0_all, relT_s, e1f_all,
             e2c2, r1c2, v1c2, e2fs2, v2c2, wbuf2, pre0c2, pre1c2,
             g1sem, g2sems, g3sems):
    wid = lax.axis_index("s") * NC + lax.axis_index("c")
    qw = u_flat.shape[0] // NW
    nch = qw // CH
    w0 = wid * qw

    pltpu.sync_copy(relT, relT_s)
    pltpu.sync_copy(u_flat.at[pl.ds(w0, qw)], uf)
    a1 = pltpu.async_copy(adj_ent.at[uf], e1_all, g1sem)
    a2 = pltpu.async_copy(adj_rel.at[uf], r0_all, g1sem)
    a3 = pltpu.async_copy(ent_table.at[uf], v0_all, g1sem)
    a1.wait(); a2.wait(); a3.wait()
    pltpu.sync_copy(r0_all, r0o.at[pl.ds(w0, qw)])
    for k in range(qw):
        e1f_all[pl.ds(k * NN, NN)] = e1_all[k]

    def g2_issue(ci, p):
        idx = e1f_all.at[pl.ds(ci * QN, QN)]
        pltpu.async_copy(adj_ent.at[idx], e2c2[p], g2sems[p])
        pltpu.async_copy(adj_rel.at[idx], r1c2[p], g2sems[p])
        pltpu.async_copy(ent_table.at[idx], v1c2[p], g2sems[p])

    def g2_wait(p):
        idx = e1f_all.at[pl.ds(0, QN)]
        pltpu.make_async_copy(adj_ent.at[idx], e2c2[p], g2sems[p]).wait()
        pltpu.make_async_copy(adj_rel.at[idx], r1c2[p], g2sems[p]).wait()
        pltpu.make_async_copy(ent_table.at[idx], v1c2[p], g2sems[p]).wait()

    def g3_issue(p):
        for k in range(NB):
            pltpu.async_copy(ent_table.at[e2fs2[p][k]],
                             v2c2[p].at[pl.ds(k * 128, 128)], g3sems[p])

    def g3_wait(p):
        for k in range(NB):
            pltpu.make_async_copy(ent_table.at[e2fs2[p][k]],
                                  v2c2[p].at[pl.ds(k * 128, 128)],
                                  g3sems[p]).wait()

    def score(r_row, va, vb):
        s0 = jnp.zeros((L,), jnp.float32)
        s1 = jnp.zeros((L,), jnp.float32)
        for d in range(L):
            s0 = s0 + plsc.load_gather(relT_s, [r_row + d * NREL]) * _bc(va[d])
            s1 = s1 + plsc.load_gather(relT_s,
                                       [r_row + (L + d) * NREL]) * _bc(vb[d])
        return s0 + s1

    def xstage(ci, p):
        r1c, v1c, wbuf, pre1c, pre0c = (
            r1c2[p], v1c2[p], wbuf2[p], pre1c2[p], pre0c2[p])

        def jbody(lr, carry2):
            r1row = r1c[lr]
            v1a = v1c[lr, 0:L]
            v1b = v1c[lr, L:DIM]
            s = score(r1row, v1a, v1b)
            m = jnp.max(s)
            e = jnp.exp(s - _bc(m))
            inv = jnp.ones((L,), jnp.float32) / _bc(jnp.sum(e))
            wbuf[lr] = e * inv
            pre1c[lr, 0:L] = v1a
            pre1c[lr, L:DIM] = v1b
            return carry2

        lax.fori_loop(0, QN, jbody, 0)

        for q in range(CH):
            qg = ci * CH + q
            r0row = r0_all[qg]
            v0a = v0_all[qg, 0:L]
            v0b = v0_all[qg, L:DIM]
            s = score(r0row, v0a, v0b)
            m = jnp.max(s)
            e = jnp.exp(s - _bc(m))
            ssum = jnp.sum(e)
            acc = [jnp.zeros((L,), jnp.float32) for _ in range(4)]
            for n in range(NN):
                wn = _bc(e[n])
                k = (n & 1) * 2
                acc[k] = acc[k] + wn * v1c[q * NN + n, 0:L]
                acc[k + 1] = acc[k + 1] + wn * v1c[q * NN + n, L:DIM]
            inv = jnp.ones((L,), jnp.float32) / _bc(ssum)
            pre0c[q, 0:L] = v0_all[qg, 0:L] + (acc[0] + acc[2]) * inv
            pre0c[q, L:DIM] = v0_all[qg, L:DIM] + (acc[1] + acc[3]) * inv

    def flatten_e2(p):
        e2c = e2c2[p]
        for k in range(QN):
            e2fs2[p][k // 8][pl.ds((k % 8) * NN, NN)] = e2c[k]

    def ystage(ci, p):
        g3_wait(p)
        v2c, wbuf, pre1c, pre0c = v2c2[p], wbuf2[p], pre1c2[p], pre0c2[p]

        def jbody(lr, carry2):
            w = wbuf[lr]
            base = lr * NN
            acc = [jnp.zeros((L,), jnp.float32) for _ in range(4)]
            for n in range(NN):
                wn = _bc(w[n])
                k = (n & 1) * 2
                acc[k] = acc[k] + wn * v2c[base + n, 0:L]
                acc[k + 1] = acc[k + 1] + wn * v2c[base + n, L:DIM]
            pre1c[lr, 0:L] = pre1c[lr, 0:L] + (acc[0] + acc[2])
            pre1c[lr, L:DIM] = pre1c[lr, L:DIM] + (acc[1] + acc[3])
            return carry2

        lax.fori_loop(0, QN, jbody, 0)
        q0 = w0 + ci * CH
        pltpu.sync_copy(pre0c, pre0.at[pl.ds(q0, CH)])
        pltpu.sync_copy(pre1c, pre1.at[pl.ds(q0 * NN, QN)])

    g2_issue(0, 0)

    def outer(io, carry):
        for p in (0, 1):
            ci = 2 * io + p
            g2_wait(p)
            xstage(ci, p)
            flatten_e2(p)
            g3_issue(p)

            @pl.when(ci < nch - 1)
            def _():
                g2_issue(ci + 1, 1 - p)

            @pl.when(ci > 0)
            def _():
                ystage(ci - 1, 1 - p)
        return carry

    lax.fori_loop(0, nch // 2, outer, 0)
    ystage(nch - 1, 1)


def _sc_call(u, adj_ent, adj_rel, ent_table, relT):
    B = u.shape[0]
    mesh = plsc.VectorSubcoreMesh(core_axis_name="c", subcore_axis_name="s",
                                  num_cores=NC, num_subcores=NS)
    qw = B // NW
    body = functools.partial(
        pl.kernel,
        out_type=(
            jax.ShapeDtypeStruct((B, DIM), jnp.float32),
            jax.ShapeDtypeStruct((B * NN, DIM), jnp.float32),
            jax.ShapeDtypeStruct((B, NN), jnp.int32),
        ),
        mesh=mesh,
        compiler_params=pltpu.CompilerParams(needs_layout_passes=False,
                                             use_tc_tiling_on_sc=False),
        scratch_types=[
            pltpu.VMEM((qw,), jnp.int32),
            pltpu.VMEM((qw, NN), jnp.int32),
            pltpu.VMEM((qw, NN), jnp.int32),
            pltpu.VMEM((qw, DIM), jnp.float32),
            pltpu.VMEM((DIM * NREL,), jnp.float32),
            pltpu.VMEM((qw * NN,), jnp.int32),
            [pltpu.VMEM((QN, NN), jnp.int32)] * 2,
            [pltpu.VMEM((QN, NN), jnp.int32)] * 2,
            [pltpu.VMEM((QN, DIM), jnp.float32)] * 2,
            [[pltpu.VMEM((128,), jnp.int32)
              for _ in range(NB)] for _ in range(2)],
            [pltpu.VMEM((CH * NN * NN, DIM), jnp.float32)] * 2,
            [pltpu.VMEM((QN, L), jnp.float32)] * 2,
            [pltpu.VMEM((CH, DIM), jnp.float32)] * 2,
            [pltpu.VMEM((QN, DIM), jnp.float32)] * 2,
            pltpu.SemaphoreType.DMA,
            [pltpu.SemaphoreType.DMA] * 2,
            [pltpu.SemaphoreType.DMA] * 2,
        ],
    )(_sc_body)
    return body(u, adj_ent, adj_rel, ent_table, relT)


BK2 = 256


def _tail_kernel(pre0_ref, pre1_ref, r0_ref, relT_ref, w_ref, b_ref, out_ref):
    bk = pre0_ref.shape[0]
    W = w_ref[...]
    b = b_ref[...]
    h0 = jax.nn.sigmoid(pre0_ref[...] @ W + b)
    h1 = jax.nn.sigmoid(pre1_ref[...] @ W + b)
    sp = h0 @ relT_ref[...]
    r0 = r0_ref[...]
    rid = lax.broadcasted_iota(jnp.int32, (bk, NN, NREL), 2)
    onehot = (r0[:, :, None] == rid).astype(jnp.float32)
    sh = jnp.sum(onehot * sp[:, None, :], axis=-1)
    sh = sh - jnp.max(sh, axis=-1, keepdims=True)
    eh = jnp.exp(sh)
    wh = eh / jnp.sum(eh, axis=-1, keepdims=True)
    h1g = h1.reshape(bk, NN, DIM)
    aggh = jnp.sum(wh[:, :, None] * h1g, axis=1)
    out_ref[...] = jnp.tanh((h0 + aggh) @ W + b)


def kernel(drug_entity_list, adj_ent, adj_rel, drug_table, ent_table, rel_table, W, b):
    B = drug_entity_list.shape[0]
    relT = rel_table.T.reshape(-1)
    pre0, pre1, r0 = _sc_call(drug_entity_list, adj_ent, adj_rel, ent_table,
                              relT)
    grid = B // BK2
    out = pl.pallas_call(
        _tail_kernel,
        grid=(grid,),
        in_specs=[
            pl.BlockSpec((BK2, DIM), lambda i: (i, 0)),
            pl.BlockSpec((BK2 * NN, DIM), lambda i: (i, 0)),
            pl.BlockSpec((BK2, NN), lambda i: (i, 0)),
            pl.BlockSpec((DIM, NREL), lambda i: (0, 0)),
            pl.BlockSpec((DIM, DIM), lambda i: (0, 0)),
            pl.BlockSpec((1, DIM), lambda i: (0, 0)),
        ],
        out_specs=pl.BlockSpec((BK2, DIM), lambda i: (i, 0)),
        out_shape=jax.ShapeDtypeStruct((B, DIM), jnp.float32),
    )(pre0, pre1, r0, rel_table.T, W, b.reshape(1, DIM))
    return out
